# baseline (speedup 1.0000x reference)
import functools

import jax
import jax.numpy as jnp
from jax import lax
from jax.experimental import pallas as pl
from jax.experimental.pallas import tpu as pltpu
from jax.experimental.pallas import tpu_sc as plsc

_N = 10000
_E = 320000
_DG = 64
_DW = 16
_NC = 2
_NS = 16
_NW = _NC * _NS
_CH = 128
_CPW = 80
_EP = _NW * _CPW * _CH
_NPAD = 16
_NA = _N + _NPAD
_RPS = 624
_TAIL0 = _RPS * _NS
_TAILN = _N - _TAIL0
_NBUF = 5
_P = 3


def _make_sc_agg(compute_deg):
  mesh = plsc.VectorSubcoreMesh(core_axis_name="c", subcore_axis_name="s")
  out_type = [jax.ShapeDtypeStruct((_NC, _N, _DG), jnp.float32)]
  scratch = [
      pltpu.VMEM((_CPW, _CH), jnp.int32),
      pltpu.VMEM((_CPW, _CH), jnp.int32),
      pltpu.VMEM((_NBUF, _CH, _DG), jnp.float32),
      pltpu.VMEM_SHARED((_NA, _DG), jnp.float32),
      pltpu.SemaphoreType.DMA((_NBUF,)),
      pltpu.SemaphoreType.DMA((_NBUF,)),
  ]
  if compute_deg:
    out_type.append(jax.ShapeDtypeStruct((_NC, _N, _DW), jnp.float32))
    scratch += [
        pltpu.VMEM((_CH, _DW), jnp.float32),
        pltpu.VMEM_SHARED((_NA, _DW), jnp.float32),
        pltpu.SemaphoreType.DMA((_NBUF,)),
    ]

  def body(*refs):
    if compute_deg:
      (vals, src, dst, z64, zdw, ones_h, out_sum, out_deg,
       srcb, dstb, rows, acc, gsem, ssem, onesb, dacc, dsem) = refs
    else:
      (vals, src, dst, z64, out_sum,
       srcb, dstb, rows, acc, gsem, ssem) = refs
    cid = lax.axis_index("c")
    sid = lax.axis_index("s")
    wid = sid * _NC + cid
    r0 = sid * _RPS
    pltpu.sync_copy(z64.at[pl.ds(r0, _RPS)], acc.at[pl.ds(r0, _RPS)])
    if compute_deg:
      pltpu.sync_copy(zdw.at[pl.ds(r0, _RPS)], dacc.at[pl.ds(r0, _RPS)])
      pltpu.sync_copy(ones_h, onesb)

    @pl.when(sid == _NS - 1)
    def _():
      pltpu.sync_copy(z64.at[pl.ds(_TAIL0, _TAILN + _NPAD)],
                      acc.at[pl.ds(_TAIL0, _TAILN + _NPAD)])
      if compute_deg:
        pltpu.sync_copy(zdw.at[pl.ds(_TAIL0, _TAILN + _NPAD)],
                        dacc.at[pl.ds(_TAIL0, _TAILN + _NPAD)])

    pltpu.sync_copy(src.at[wid], srcb)
    pltpu.sync_copy(dst.at[wid], dstb)
    plsc.subcore_barrier()

    for b in range(_P):
      pltpu.async_copy(vals.at[srcb.at[b]], rows.at[b], gsem.at[b])

    @pl.loop(0, _CPW, step=_NBUF)
    def _(c):
      for b in range(_NBUF):
        cc = c + b
        pltpu.make_async_copy(vals.at[srcb.at[cc]], rows.at[b],
                              gsem.at[b]).wait()
        pltpu.async_copy(rows.at[b], acc.at[dstb.at[cc]], ssem.at[b],
                         add=True)
        if compute_deg:
          @pl.when(cc >= _NBUF)
          def _():
            pltpu.make_async_copy(onesb, dacc.at[dstb.at[cc]],
                                  dsem.at[b]).wait()
          pltpu.async_copy(onesb, dacc.at[dstb.at[cc]], dsem.at[b],
                           add=True)
        bn = (b + _P) % _NBUF

        @pl.when(jnp.logical_and(cc + _P < _CPW, cc + _P >= _NBUF))
        def _():
          pltpu.make_async_copy(rows.at[bn], acc.at[dstb.at[cc]],
                                ssem.at[bn]).wait()

        @pl.when(cc + _P < _CPW)
        def _():
          pltpu.async_copy(vals.at[srcb.at[cc + _P]], rows.at[bn],
                           gsem.at[bn])

    for b in range(_NBUF):
      pltpu.make_async_copy(rows.at[b], acc.at[dstb.at[0]],
                            ssem.at[b]).wait()
      if compute_deg:
        pltpu.make_async_copy(onesb, dacc.at[dstb.at[0]],
                              dsem.at[b]).wait()

    plsc.subcore_barrier()
    pltpu.sync_copy(acc.at[pl.ds(r0, _RPS)],
                    out_sum.at[cid].at[pl.ds(r0, _RPS)])
    if compute_deg:
      pltpu.sync_copy(dacc.at[pl.ds(r0, _RPS)],
                      out_deg.at[cid].at[pl.ds(r0, _RPS)])

    @pl.when(sid == _NS - 1)
    def _():
      pltpu.sync_copy(acc.at[pl.ds(_TAIL0, _TAILN)],
                      out_sum.at[cid].at[pl.ds(_TAIL0, _TAILN)])
      if compute_deg:
        pltpu.sync_copy(dacc.at[pl.ds(_TAIL0, _TAILN)],
                        out_deg.at[cid].at[pl.ds(_TAIL0, _TAILN)])

  return pl.kernel(
      body,
      out_type=tuple(out_type) if compute_deg else out_type[0],
      mesh=mesh,
      scratch_types=scratch,
      compiler_params=pltpu.CompilerParams(use_tc_tiling_on_sc=False),
  )


_sc_agg_deg = _make_sc_agg(True)
_sc_agg = _make_sc_agg(False)

_R = 1000


def _tc_conv1_body(sa, sb, da, db, xb, wb, bb, ob):
  s = sa[...] + sb[...]
  deg = da[...] + db[...]
  inv = 1.0 / jnp.maximum(deg[:, 0:1], 1.0)
  agg = jnp.dot(s * inv, wb[...], preferred_element_type=jnp.float32)
  ob[...] = xb[...] + agg + bb[...]


def _tc_conv1(sa, sb, da, db, x0, W0, b0):
  grid = (_N // _R,)
  row = lambda i: (i, 0)
  fixed = lambda i: (0, 0)
  return pl.pallas_call(
      _tc_conv1_body,
      grid=grid,
      in_specs=[
          pl.BlockSpec((_R, _DG), row),
          pl.BlockSpec((_R, _DG), row),
          pl.BlockSpec((_R, _DW), row),
          pl.BlockSpec((_R, _DW), row),
          pl.BlockSpec((_R, _DG), row),
          pl.BlockSpec((_DG, _DG), fixed),
          pl.BlockSpec((1, _DG), fixed),
      ],
      out_specs=pl.BlockSpec((_R, _DG), row),
      out_shape=jax.ShapeDtypeStruct((_N, _DG), jnp.float32),
  )(sa, sb, da, db, x0, W0, b0)


def _tc_conv2_body(sa, sb, da, db, xb, y0b, wb, bb, ob):
  s = sa[...] + sb[...]
  deg = da[...] + db[...]
  inv = 1.0 / jnp.maximum(deg[:, 0:1], 1.0)
  agg = jnp.dot(s * inv, wb[...], preferred_element_type=jnp.float32)
  y1 = xb[...] + agg + bb[...]
  ob[...] = jnp.concatenate([y0b[...], y1], axis=1)


def _tc_conv2(sa, sb, da, db, x1, y0, W1, b1):
  grid = (_N // _R,)
  row = lambda i: (i, 0)
  fixed = lambda i: (0, 0)
  return pl.pallas_call(
      _tc_conv2_body,
      grid=grid,
      in_specs=[
          pl.BlockSpec((_R, _DG), row),
          pl.BlockSpec((_R, _DG), row),
          pl.BlockSpec((_R, _DW), row),
          pl.BlockSpec((_R, _DW), row),
          pl.BlockSpec((_R, _DG), row),
          pl.BlockSpec((_R, _DG), row),
          pl.BlockSpec((_DG, _DG), fixed),
          pl.BlockSpec((1, _DG), fixed),
      ],
      out_specs=pl.BlockSpec((_R, 2 * _DG), row),
      out_shape=jax.ShapeDtypeStruct((_N, 2 * _DG), jnp.float32),
  )(sa, sb, da, db, x1, y0, W1, b1)


def kernel(x, edge_index, W0, b0, W1, b1):
  npad = _EP - _E
  pad_src = jnp.arange(npad, dtype=jnp.int32) % _N
  pad_dst = _N + (jnp.arange(npad, dtype=jnp.int32) % _NPAD)
  src = jnp.concatenate([edge_index[0], pad_src]).reshape(_NW, _CPW, _CH)
  dst = jnp.concatenate([edge_index[1], pad_dst]).reshape(_NW, _CPW, _CH)
  x0 = x[:, :_DG]
  x1 = x[:, _DG:]
  z64 = jnp.zeros((_NA, _DG), jnp.float32)
  zdw = jnp.zeros((_NA, _DW), jnp.float32)
  ones = jnp.ones((_CH, _DW), jnp.float32)
  b0r = b0.reshape(1, _DG)
  b1r = b1.reshape(1, _DG)

  s0, degp = _sc_agg_deg(x1, src, dst, z64, zdw, ones)
  y0 = _tc_conv1(s0[0], s0[1], degp[0], degp[1], x0, W0, b0r)
  s1 = _sc_agg(y0, src, dst, z64)
  return _tc_conv2(s1[0], s1[1], degp[0], degp[1], x1, y0, W1, b1r)


# speedup vs baseline: 1.1246x; 1.1246x over previous
import functools

import jax
import jax.numpy as jnp
from jax import lax
from jax.experimental import pallas as pl
from jax.experimental.pallas import tpu as pltpu
from jax.experimental.pallas import tpu_sc as plsc

_N = 10000
_E = 320000
_DG = 64
_DW = 16
_NC = 2
_NS = 16
_NW = _NC * _NS
_CH = 128
_CPW = 80
_EP = _NW * _CPW * _CH
_NPAD = 16
_NA = _N + _NPAD
_RPS = 624
_TAIL0 = _RPS * _NS
_TAILN = _N - _TAIL0
_NBUF = 5
_P = 3


def _make_sc_agg(compute_deg):
  mesh = plsc.VectorSubcoreMesh(core_axis_name="c", subcore_axis_name="s")
  out_type = [jax.ShapeDtypeStruct((_NC, _N, _DG), jnp.float32)]
  scratch = [
      pltpu.VMEM((_CPW, _CH), jnp.int32),
      pltpu.VMEM((_CPW, _CH), jnp.int32),
      pltpu.VMEM((_NBUF, _CH, _DG), jnp.float32),
      pltpu.VMEM_SHARED((_NA, _DG), jnp.float32),
      pltpu.SemaphoreType.DMA((_NBUF,)),
      pltpu.SemaphoreType.DMA((_NBUF,)),
  ]
  if compute_deg:
    out_type.append(jax.ShapeDtypeStruct((_NC, _N, _DW), jnp.float32))
    scratch += [
        pltpu.VMEM((_CH, _DW), jnp.float32),
        pltpu.VMEM_SHARED((_NA, _DW), jnp.float32),
        pltpu.SemaphoreType.DMA((_NBUF,)),
    ]

  def body(*refs):
    if compute_deg:
      (vals, src, dst, z64, zdw, ones_h, out_sum, out_deg,
       srcb, dstb, rows, acc, gsem, ssem, onesb, dacc, dsem) = refs
    else:
      (vals, src, dst, z64, out_sum,
       srcb, dstb, rows, acc, gsem, ssem) = refs
    cid = lax.axis_index("c")
    sid = lax.axis_index("s")
    wid = sid * _NC + cid
    r0 = sid * _RPS
    pltpu.sync_copy(z64.at[pl.ds(r0, _RPS)], acc.at[pl.ds(r0, _RPS)])
    if compute_deg:
      pltpu.sync_copy(zdw.at[pl.ds(r0, _RPS)], dacc.at[pl.ds(r0, _RPS)])
      pltpu.sync_copy(ones_h, onesb)

    @pl.when(sid == _NS - 1)
    def _():
      pltpu.sync_copy(z64.at[pl.ds(_TAIL0, _TAILN + _NPAD)],
                      acc.at[pl.ds(_TAIL0, _TAILN + _NPAD)])
      if compute_deg:
        pltpu.sync_copy(zdw.at[pl.ds(_TAIL0, _TAILN + _NPAD)],
                        dacc.at[pl.ds(_TAIL0, _TAILN + _NPAD)])

    pltpu.sync_copy(src.at[wid], srcb)
    pltpu.sync_copy(dst.at[wid], dstb)
    plsc.subcore_barrier()

    for b in range(_P):
      pltpu.async_copy(vals.at[srcb.at[b]], rows.at[b], gsem.at[b])

    @pl.loop(0, _CPW, step=_NBUF)
    def _(c):
      for b in range(_NBUF):
        cc = c + b
        pltpu.make_async_copy(vals.at[srcb.at[cc]], rows.at[b],
                              gsem.at[b]).wait()
        pltpu.async_copy(rows.at[b], acc.at[dstb.at[cc]], ssem.at[b],
                         add=True)
        if compute_deg:
          @pl.when(cc >= _NBUF)
          def _():
            pltpu.make_async_copy(onesb, dacc.at[dstb.at[cc]],
                                  dsem.at[b]).wait()
          pltpu.async_copy(onesb, dacc.at[dstb.at[cc]], dsem.at[b],
                           add=True)
        bn = (b + _P) % _NBUF

        @pl.when(jnp.logical_and(cc + _P < _CPW, cc + _P >= _NBUF))
        def _():
          pltpu.make_async_copy(rows.at[bn], acc.at[dstb.at[cc]],
                                ssem.at[bn]).wait()

        @pl.when(cc + _P < _CPW)
        def _():
          pltpu.async_copy(vals.at[srcb.at[cc + _P]], rows.at[bn],
                           gsem.at[bn])

    for b in range(_NBUF):
      pltpu.make_async_copy(rows.at[b], acc.at[dstb.at[0]],
                            ssem.at[b]).wait()
      if compute_deg:
        pltpu.make_async_copy(onesb, dacc.at[dstb.at[0]],
                              dsem.at[b]).wait()

    plsc.subcore_barrier()
    pltpu.sync_copy(acc.at[pl.ds(r0, _RPS)],
                    out_sum.at[cid].at[pl.ds(r0, _RPS)])
    if compute_deg:
      pltpu.sync_copy(dacc.at[pl.ds(r0, _RPS)],
                      out_deg.at[cid].at[pl.ds(r0, _RPS)])

    @pl.when(sid == _NS - 1)
    def _():
      pltpu.sync_copy(acc.at[pl.ds(_TAIL0, _TAILN)],
                      out_sum.at[cid].at[pl.ds(_TAIL0, _TAILN)])
      if compute_deg:
        pltpu.sync_copy(dacc.at[pl.ds(_TAIL0, _TAILN)],
                        out_deg.at[cid].at[pl.ds(_TAIL0, _TAILN)])

  return pl.kernel(
      body,
      out_type=tuple(out_type) if compute_deg else out_type[0],
      mesh=mesh,
      scratch_types=scratch,
      compiler_params=pltpu.CompilerParams(use_tc_tiling_on_sc=False),
  )


_sc_agg_deg = _make_sc_agg(True)
_sc_agg = _make_sc_agg(False)

_R = 2000


def _tc_conv1_body(s, deg, xb, wb, bb, ob):
  ssum = s[0] + s[1]
  d = deg[0] + deg[1]
  inv = 1.0 / jnp.maximum(d[:, 0:1], 1.0)
  agg = jnp.dot(ssum * inv, wb[...], preferred_element_type=jnp.float32)
  ob[...] = xb[:, :_DG] + agg + bb[...]


def _tc_conv1(s0, degp, x, W0, b0):
  grid = (_N // _R,)
  return pl.pallas_call(
      _tc_conv1_body,
      grid=grid,
      in_specs=[
          pl.BlockSpec((2, _R, _DG), lambda i: (0, i, 0)),
          pl.BlockSpec((2, _R, _DW), lambda i: (0, i, 0)),
          pl.BlockSpec((_R, 2 * _DG), lambda i: (i, 0)),
          pl.BlockSpec((_DG, _DG), lambda i: (0, 0)),
          pl.BlockSpec((1, _DG), lambda i: (0, 0)),
      ],
      out_specs=pl.BlockSpec((_R, _DG), lambda i: (i, 0)),
      out_shape=jax.ShapeDtypeStruct((_N, _DG), jnp.float32),
  )(s0, degp, x, W0, b0)


def _tc_conv2_body(s, deg, xb, y0b, wb, bb, ob):
  ssum = s[0] + s[1]
  d = deg[0] + deg[1]
  inv = 1.0 / jnp.maximum(d[:, 0:1], 1.0)
  agg = jnp.dot(ssum * inv, wb[...], preferred_element_type=jnp.float32)
  y1 = xb[:, _DG:] + agg + bb[...]
  ob[...] = jnp.concatenate([y0b[...], y1], axis=1)


def _tc_conv2(s1, degp, x, y0, W1, b1):
  grid = (_N // _R,)
  return pl.pallas_call(
      _tc_conv2_body,
      grid=grid,
      in_specs=[
          pl.BlockSpec((2, _R, _DG), lambda i: (0, i, 0)),
          pl.BlockSpec((2, _R, _DW), lambda i: (0, i, 0)),
          pl.BlockSpec((_R, 2 * _DG), lambda i: (i, 0)),
          pl.BlockSpec((_R, _DG), lambda i: (i, 0)),
          pl.BlockSpec((_DG, _DG), lambda i: (0, 0)),
          pl.BlockSpec((1, _DG), lambda i: (0, 0)),
      ],
      out_specs=pl.BlockSpec((_R, 2 * _DG), lambda i: (i, 0)),
      out_shape=jax.ShapeDtypeStruct((_N, 2 * _DG), jnp.float32),
  )(s1, degp, x, y0, W1, b1)


def kernel(x, edge_index, W0, b0, W1, b1):
  npad = _EP - _E
  pad_src = jnp.arange(npad, dtype=jnp.int32) % _N
  pad_dst = _N + (jnp.arange(npad, dtype=jnp.int32) % _NPAD)
  src = jnp.concatenate([edge_index[0], pad_src]).reshape(_NW, _CPW, _CH)
  dst = jnp.concatenate([edge_index[1], pad_dst]).reshape(_NW, _CPW, _CH)
  x1 = x[:, _DG:]
  z64 = jnp.zeros((_NA, _DG), jnp.float32)
  zdw = jnp.zeros((_NA, _DW), jnp.float32)
  ones = jnp.ones((_CH, _DW), jnp.float32)
  b0r = b0.reshape(1, _DG)
  b1r = b1.reshape(1, _DG)

  s0, degp = _sc_agg_deg(x1, src, dst, z64, zdw, ones)
  y0 = _tc_conv1(s0, degp, x, W0, b0r)
  s1 = _sc_agg(y0, src, dst, z64)
  return _tc_conv2(s1, degp, x, y0, W1, b1r)


# speedup vs baseline: 1.1891x; 1.0573x over previous
import functools

import jax
import jax.numpy as jnp
from jax import lax
from jax.experimental import pallas as pl
from jax.experimental.pallas import tpu as pltpu
from jax.experimental.pallas import tpu_sc as plsc

_N = 10000
_E = 320000
_DG = 64
_DW = 16
_NC = 2
_NS = 16
_NW = _NC * _NS
_CH = 128
_CPW = 80
_EP = _NW * _CPW * _CH
_NPAD = 16
_NA = _N + _NPAD
_RPS = 624
_TAIL0 = _RPS * _NS
_TAILN = _N - _TAIL0
_NBUF = 5
_P = 3


def _make_sc_agg(compute_deg):
  mesh = plsc.VectorSubcoreMesh(core_axis_name="c", subcore_axis_name="s")
  out_type = [jax.ShapeDtypeStruct((_NC, _N, _DG), jnp.float32)]
  scratch = [
      pltpu.VMEM((_CPW, _CH), jnp.int32),
      pltpu.VMEM((_CPW, _CH), jnp.int32),
      pltpu.VMEM((_NBUF, _CH, _DG), jnp.float32),
      pltpu.VMEM_SHARED((_NA, _DG), jnp.float32),
      pltpu.SemaphoreType.DMA((_NBUF,)),
      pltpu.SemaphoreType.DMA((_NBUF,)),
  ]
  if compute_deg:
    out_type.append(jax.ShapeDtypeStruct((_NC, _N, _DW), jnp.float32))
    scratch += [
        pltpu.VMEM((_CH, _DW), jnp.float32),
        pltpu.VMEM_SHARED((_NA, _DW), jnp.float32),
        pltpu.SemaphoreType.DMA((_NBUF,)),
    ]

  def body(*refs):
    if compute_deg:
      (vals, src, dst, z64, zdw, ones_h, out_sum, out_deg,
       srcb, dstb, rows, acc, gsem, ssem, onesb, dacc, dsem) = refs
    else:
      (vals, src, dst, z64, out_sum,
       srcb, dstb, rows, acc, gsem, ssem) = refs
    cid = lax.axis_index("c")
    sid = lax.axis_index("s")
    wid = sid * _NC + cid
    r0 = sid * _RPS
    pltpu.sync_copy(z64.at[pl.ds(r0, _RPS)], acc.at[pl.ds(r0, _RPS)])
    if compute_deg:
      pltpu.sync_copy(zdw.at[pl.ds(r0, _RPS)], dacc.at[pl.ds(r0, _RPS)])
      pltpu.sync_copy(ones_h, onesb)

    @pl.when(sid == _NS - 1)
    def _():
      pltpu.sync_copy(z64.at[pl.ds(_TAIL0, _TAILN + _NPAD)],
                      acc.at[pl.ds(_TAIL0, _TAILN + _NPAD)])
      if compute_deg:
        pltpu.sync_copy(zdw.at[pl.ds(_TAIL0, _TAILN + _NPAD)],
                        dacc.at[pl.ds(_TAIL0, _TAILN + _NPAD)])

    pltpu.sync_copy(src.at[wid], srcb)
    pltpu.sync_copy(dst.at[wid], dstb)
    plsc.subcore_barrier()

    for b in range(_P):
      pltpu.async_copy(vals.at[srcb.at[b]], rows.at[b], gsem.at[b])

    @pl.loop(0, _CPW, step=_NBUF)
    def _(c):
      for b in range(_NBUF):
        cc = c + b
        pltpu.make_async_copy(vals.at[srcb.at[cc]], rows.at[b],
                              gsem.at[b]).wait()
        pltpu.async_copy(rows.at[b], acc.at[dstb.at[cc]], ssem.at[b],
                         add=True)
        if compute_deg:
          @pl.when(cc >= _NBUF)
          def _():
            pltpu.make_async_copy(onesb, dacc.at[dstb.at[cc]],
                                  dsem.at[b]).wait()
          pltpu.async_copy(onesb, dacc.at[dstb.at[cc]], dsem.at[b],
                           add=True)
        bn = (b + _P) % _NBUF

        @pl.when(jnp.logical_and(cc + _P < _CPW, cc + _P >= _NBUF))
        def _():
          pltpu.make_async_copy(rows.at[bn], acc.at[dstb.at[cc]],
                                ssem.at[bn]).wait()

        @pl.when(cc + _P < _CPW)
        def _():
          pltpu.async_copy(vals.at[srcb.at[cc + _P]], rows.at[bn],
                           gsem.at[bn])

    for b in range(_NBUF):
      pltpu.make_async_copy(rows.at[b], acc.at[dstb.at[0]],
                            ssem.at[b]).wait()
      if compute_deg:
        pltpu.make_async_copy(onesb, dacc.at[dstb.at[0]],
                              dsem.at[b]).wait()

    plsc.subcore_barrier()
    pltpu.sync_copy(acc.at[pl.ds(r0, _RPS)],
                    out_sum.at[cid].at[pl.ds(r0, _RPS)])
    if compute_deg:
      pltpu.sync_copy(dacc.at[pl.ds(r0, _RPS)],
                      out_deg.at[cid].at[pl.ds(r0, _RPS)])

    @pl.when(sid == _NS - 1)
    def _():
      pltpu.sync_copy(acc.at[pl.ds(_TAIL0, _TAILN)],
                      out_sum.at[cid].at[pl.ds(_TAIL0, _TAILN)])
      if compute_deg:
        pltpu.sync_copy(dacc.at[pl.ds(_TAIL0, _TAILN)],
                        out_deg.at[cid].at[pl.ds(_TAIL0, _TAILN)])

  return pl.kernel(
      body,
      out_type=tuple(out_type) if compute_deg else out_type[0],
      mesh=mesh,
      scratch_types=scratch,
      compiler_params=pltpu.CompilerParams(use_tc_tiling_on_sc=False),
  )


_sc_agg_deg = _make_sc_agg(True)
_sc_agg = _make_sc_agg(False)

_R = 2000


def _tc_conv1_body(s, deg, xb, wb, bb, ob):
  ssum = s[0] + s[1]
  d = deg[0] + deg[1]
  inv = 1.0 / jnp.maximum(d[:, 0:1], 1.0)
  agg = jnp.dot(ssum * inv, wb[...], preferred_element_type=jnp.float32)
  y = xb[:, :_DG] + agg + bb[...]
  ob[...] = jnp.concatenate([y, jnp.zeros_like(y)], axis=1)


def _tc_conv1(s0, degp, x, W0, b0):
  grid = (_N // _R,)
  return pl.pallas_call(
      _tc_conv1_body,
      grid=grid,
      in_specs=[
          pl.BlockSpec((2, _R, _DG), lambda i: (0, i, 0)),
          pl.BlockSpec((2, _R, _DW), lambda i: (0, i, 0)),
          pl.BlockSpec((_R, 2 * _DG), lambda i: (i, 0)),
          pl.BlockSpec((_DG, _DG), lambda i: (0, 0)),
          pl.BlockSpec((1, _DG), lambda i: (0, 0)),
      ],
      out_specs=pl.BlockSpec((_R, 2 * _DG), lambda i: (i, 0)),
      out_shape=jax.ShapeDtypeStruct((_N, 2 * _DG), jnp.float32),
  )(s0, degp, x, W0, b0)


def _tc_conv2_body(s, deg, xb, y0b, wb, bb, ob):
  ssum = s[0] + s[1]
  d = deg[0] + deg[1]
  inv = 1.0 / jnp.maximum(d[:, 0:1], 1.0)
  agg = jnp.dot(ssum * inv, wb[...], preferred_element_type=jnp.float32)
  y1 = xb[:, _DG:] + agg + bb[...]
  ob[...] = jnp.concatenate([y0b[:, :_DG], y1], axis=1)


def _tc_conv2(s1, degp, x, y0, W1, b1):
  grid = (_N // _R,)
  return pl.pallas_call(
      _tc_conv2_body,
      grid=grid,
      in_specs=[
          pl.BlockSpec((2, _R, _DG), lambda i: (0, i, 0)),
          pl.BlockSpec((2, _R, _DW), lambda i: (0, i, 0)),
          pl.BlockSpec((_R, 2 * _DG), lambda i: (i, 0)),
          pl.BlockSpec((_R, 2 * _DG), lambda i: (i, 0)),
          pl.BlockSpec((_DG, _DG), lambda i: (0, 0)),
          pl.BlockSpec((1, _DG), lambda i: (0, 0)),
      ],
      out_specs=pl.BlockSpec((_R, 2 * _DG), lambda i: (i, 0)),
      out_shape=jax.ShapeDtypeStruct((_N, 2 * _DG), jnp.float32),
  )(s1, degp, x, y0, W1, b1)


def kernel(x, edge_index, W0, b0, W1, b1):
  npad = _EP - _E
  pad_src = jnp.arange(npad, dtype=jnp.int32) % _N
  pad_dst = _N + (jnp.arange(npad, dtype=jnp.int32) % _NPAD)
  src = jnp.concatenate([edge_index[0], pad_src])
  src1 = (2 * src + 1).reshape(_NW, _CPW, _CH)
  src2 = (2 * src).reshape(_NW, _CPW, _CH)
  dst = jnp.concatenate([edge_index[1], pad_dst]).reshape(_NW, _CPW, _CH)
  z64 = jnp.zeros((_NA, _DG), jnp.float32)
  zdw = jnp.zeros((_NA, _DW), jnp.float32)
  ones = jnp.ones((_CH, _DW), jnp.float32)
  b0r = b0.reshape(1, _DG)
  b1r = b1.reshape(1, _DG)

  x2 = x.reshape(2 * _N, _DG)
  s0, degp = _sc_agg_deg(x2, src1, dst, z64, zdw, ones)
  y0 = _tc_conv1(s0, degp, x, W0, b0r)
  s1 = _sc_agg(y0.reshape(2 * _N, _DG), src2, dst, z64)
  return _tc_conv2(s1, degp, x, y0, W1, b1r)


# speedup vs baseline: 1.1897x; 1.0005x over previous
import functools

import jax
import jax.numpy as jnp
from jax import lax
from jax.experimental import pallas as pl
from jax.experimental.pallas import tpu as pltpu
from jax.experimental.pallas import tpu_sc as plsc

_N = 10000
_E = 320000
_DG = 64
_DW = 16
_NC = 2
_NS = 16
_NW = _NC * _NS
_CH = 128
_CPW = 80
_EP = _NW * _CPW * _CH
_NPAD = 16
_NA = _N + _NPAD
_RPS = 624
_TAIL0 = _RPS * _NS
_TAILN = _N - _TAIL0
_NBUF = 5
_P = 3


def _make_sc_agg(compute_deg):
  mesh = plsc.VectorSubcoreMesh(core_axis_name="c", subcore_axis_name="s")
  out_type = [jax.ShapeDtypeStruct((_NC, _N, _DG), jnp.float32)]
  scratch = [
      pltpu.VMEM((_CPW, _CH), jnp.int32),
      pltpu.VMEM((_CPW, _CH), jnp.int32),
      pltpu.VMEM((_NBUF, _CH, _DG), jnp.float32),
      pltpu.VMEM_SHARED((_NA, _DG), jnp.float32),
      pltpu.SemaphoreType.DMA((_NBUF,)),
      pltpu.SemaphoreType.DMA((_NBUF,)),
  ]
  if compute_deg:
    out_type.append(jax.ShapeDtypeStruct((_NC, _N, _DW), jnp.float32))
    scratch += [
        pltpu.VMEM((_CH, _DW), jnp.float32),
        pltpu.VMEM_SHARED((_NA, _DW), jnp.float32),
        pltpu.SemaphoreType.DMA((_NBUF,)),
    ]

  def body(*refs):
    if compute_deg:
      (vals, src, dst, z64, zdw, ones_h, out_sum, out_deg,
       srcb, dstb, rows, acc, gsem, ssem, onesb, dacc, dsem) = refs
    else:
      (vals, src, dst, z64, out_sum,
       srcb, dstb, rows, acc, gsem, ssem) = refs
    cid = lax.axis_index("c")
    sid = lax.axis_index("s")
    wid = sid * _NC + cid
    r0 = sid * _RPS
    pltpu.sync_copy(z64.at[pl.ds(r0, _RPS)], acc.at[pl.ds(r0, _RPS)])
    if compute_deg:
      pltpu.sync_copy(zdw.at[pl.ds(r0, _RPS)], dacc.at[pl.ds(r0, _RPS)])
      pltpu.sync_copy(ones_h, onesb)

    @pl.when(sid == _NS - 1)
    def _():
      pltpu.sync_copy(z64.at[pl.ds(_TAIL0, _TAILN + _NPAD)],
                      acc.at[pl.ds(_TAIL0, _TAILN + _NPAD)])
      if compute_deg:
        pltpu.sync_copy(zdw.at[pl.ds(_TAIL0, _TAILN + _NPAD)],
                        dacc.at[pl.ds(_TAIL0, _TAILN + _NPAD)])

    pltpu.sync_copy(src.at[wid], srcb)
    pltpu.sync_copy(dst.at[wid], dstb)
    plsc.subcore_barrier()

    for b in range(_P):
      pltpu.async_copy(vals.at[srcb.at[b]], rows.at[b], gsem.at[b])

    @pl.loop(0, _CPW, step=_NBUF)
    def _(c):
      for b in range(_NBUF):
        cc = c + b
        pltpu.make_async_copy(vals.at[srcb.at[cc]], rows.at[b],
                              gsem.at[b]).wait()
        pltpu.async_copy(rows.at[b], acc.at[dstb.at[cc]], ssem.at[b],
                         add=True)
        if compute_deg:
          @pl.when(cc >= _NBUF)
          def _():
            pltpu.make_async_copy(onesb, dacc.at[dstb.at[cc - _NBUF]],
                                  dsem.at[b]).wait()
          pltpu.async_copy(onesb, dacc.at[dstb.at[cc]], dsem.at[b],
                           add=True)
        bn = (b + _P) % _NBUF

        @pl.when(jnp.logical_and(cc + _P < _CPW, cc + _P >= _NBUF))
        def _():
          pltpu.make_async_copy(rows.at[bn], acc.at[dstb.at[cc + _P - _NBUF]],
                                ssem.at[bn]).wait()

        @pl.when(cc + _P < _CPW)
        def _():
          pltpu.async_copy(vals.at[srcb.at[cc + _P]], rows.at[bn],
                           gsem.at[bn])

    for b in range(_NBUF):
      pltpu.make_async_copy(rows.at[b], acc.at[dstb.at[_CPW - _NBUF + b]],
                            ssem.at[b]).wait()
      if compute_deg:
        pltpu.make_async_copy(onesb, dacc.at[dstb.at[_CPW - _NBUF + b]],
                              dsem.at[b]).wait()

    plsc.subcore_barrier()
    pltpu.sync_copy(acc.at[pl.ds(r0, _RPS)],
                    out_sum.at[cid].at[pl.ds(r0, _RPS)])
    if compute_deg:
      pltpu.sync_copy(dacc.at[pl.ds(r0, _RPS)],
                      out_deg.at[cid].at[pl.ds(r0, _RPS)])

    @pl.when(sid == _NS - 1)
    def _():
      pltpu.sync_copy(acc.at[pl.ds(_TAIL0, _TAILN)],
                      out_sum.at[cid].at[pl.ds(_TAIL0, _TAILN)])
      if compute_deg:
        pltpu.sync_copy(dacc.at[pl.ds(_TAIL0, _TAILN)],
                        out_deg.at[cid].at[pl.ds(_TAIL0, _TAILN)])

  return pl.kernel(
      body,
      out_type=tuple(out_type) if compute_deg else out_type[0],
      mesh=mesh,
      scratch_types=scratch,
      compiler_params=pltpu.CompilerParams(use_tc_tiling_on_sc=False),
  )


_sc_agg_deg = _make_sc_agg(True)
_sc_agg = _make_sc_agg(False)

_R = 2000


def _tc_conv1_body(s, deg, xb, wb, bb, ob):
  ssum = s[0] + s[1]
  d = deg[0] + deg[1]
  inv = 1.0 / jnp.maximum(d[:, 0:1], 1.0)
  agg = jnp.dot(ssum * inv, wb[...], preferred_element_type=jnp.float32)
  y = xb[:, :_DG] + agg + bb[...]
  ob[...] = jnp.concatenate([y, jnp.zeros_like(y)], axis=1)


def _tc_conv1(s0, degp, x, W0, b0):
  grid = (_N // _R,)
  return pl.pallas_call(
      _tc_conv1_body,
      grid=grid,
      in_specs=[
          pl.BlockSpec((2, _R, _DG), lambda i: (0, i, 0)),
          pl.BlockSpec((2, _R, _DW), lambda i: (0, i, 0)),
          pl.BlockSpec((_R, 2 * _DG), lambda i: (i, 0)),
          pl.BlockSpec((_DG, _DG), lambda i: (0, 0)),
          pl.BlockSpec((1, _DG), lambda i: (0, 0)),
      ],
      out_specs=pl.BlockSpec((_R, 2 * _DG), lambda i: (i, 0)),
      out_shape=jax.ShapeDtypeStruct((_N, 2 * _DG), jnp.float32),
  )(s0, degp, x, W0, b0)


def _tc_conv2_body(s, deg, xb, y0b, wb, bb, ob):
  ssum = s[0] + s[1]
  d = deg[0] + deg[1]
  inv = 1.0 / jnp.maximum(d[:, 0:1], 1.0)
  agg = jnp.dot(ssum * inv, wb[...], preferred_element_type=jnp.float32)
  y1 = xb[:, _DG:] + agg + bb[...]
  ob[...] = jnp.concatenate([y0b[:, :_DG], y1], axis=1)


def _tc_conv2(s1, degp, x, y0, W1, b1):
  grid = (_N // _R,)
  return pl.pallas_call(
      _tc_conv2_body,
      grid=grid,
      in_specs=[
          pl.BlockSpec((2, _R, _DG), lambda i: (0, i, 0)),
          pl.BlockSpec((2, _R, _DW), lambda i: (0, i, 0)),
          pl.BlockSpec((_R, 2 * _DG), lambda i: (i, 0)),
          pl.BlockSpec((_R, 2 * _DG), lambda i: (i, 0)),
          pl.BlockSpec((_DG, _DG), lambda i: (0, 0)),
          pl.BlockSpec((1, _DG), lambda i: (0, 0)),
      ],
      out_specs=pl.BlockSpec((_R, 2 * _DG), lambda i: (i, 0)),
      out_shape=jax.ShapeDtypeStruct((_N, 2 * _DG), jnp.float32),
  )(s1, degp, x, y0, W1, b1)


def kernel(x, edge_index, W0, b0, W1, b1):
  npad = _EP - _E
  pad_src = jnp.arange(npad, dtype=jnp.int32) % _N
  pad_dst = _N + (jnp.arange(npad, dtype=jnp.int32) % _NPAD)
  src = jnp.concatenate([edge_index[0], pad_src])
  src1 = (2 * src + 1).reshape(_NW, _CPW, _CH)
  src2 = (2 * src).reshape(_NW, _CPW, _CH)
  dst = jnp.concatenate([edge_index[1], pad_dst]).reshape(_NW, _CPW, _CH)
  z64 = jnp.zeros((_NA, _DG), jnp.float32)
  zdw = jnp.zeros((_NA, _DW), jnp.float32)
  ones = jnp.ones((_CH, _DW), jnp.float32)
  b0r = b0.reshape(1, _DG)
  b1r = b1.reshape(1, _DG)

  x2 = x.reshape(2 * _N, _DG)
  s0, degp = _sc_agg_deg(x2, src1, dst, z64, zdw, ones)
  y0 = _tc_conv1(s0, degp, x, W0, b0r)
  s1 = _sc_agg(y0.reshape(2 * _N, _DG), src2, dst, z64)
  return _tc_conv2(s1, degp, x, y0, W1, b1r)
